# baseline (speedup 1.0000x reference)
import functools

import jax
import jax.numpy as jnp
from jax import lax
from jax.experimental import pallas as pl
from jax.experimental.pallas import tpu as pltpu
from jax.experimental.pallas import tpu_sc as plsc

F32 = jnp.float32
I32 = jnp.int32

N_ = 10000
E_ = 320000
P_ = 200000
HD_ = 64
E1 = E_ + N_

NC, NS = 2, 16
NW = NC * NS
CHUNK = 128
D_ = 128

EC = 82
EW = CHUNK * EC
EP = NW * EW
EC2 = 2 * EC
EW2 = CHUNK * EC2

PC = 50
PW = CHUNK * PC
PP = NW * PW

NP = 10240
RT = NP // NS
ZR = 128


def _mesh():
    return plsc.VectorSubcoreMesh(core_axis_name="c", subcore_axis_name="s")


def _sc_gather2(table_a, table_b, idx_a, idx_b, per_w, n_chunks, total):
    @functools.partial(
        pl.kernel,
        out_type=(jax.ShapeDtypeStruct((total, D_), F32),
                  jax.ShapeDtypeStruct((total, D_), F32)),
        mesh=_mesh(),
        scratch_types=[
            pltpu.VMEM((2, CHUNK), I32),
            pltpu.VMEM((2, CHUNK), I32),
            pltpu.VMEM((2, CHUNK, D_), F32),
            pltpu.VMEM((2, CHUNK, D_), F32),
        ] + [pltpu.SemaphoreType.DMA] * 12,
    )
    def k(ta2, tb2, ia, ib, oa, ob, ia_v, ib_v, ra_v, rb_v, *sems):
        sia, sib = sems[0:2], sems[2:4]
        sga, sgb = sems[4:6], sems[6:8]
        sta, stb = sems[8:10], sems[10:12]
        c = lax.axis_index("c")
        wid = lax.axis_index("s") * NC + c
        base = wid * per_w
        ta = ta2.at[c]
        tb = tb2.at[c]

        for b in range(2):
            off = base + b * CHUNK
            pltpu.async_copy(ia.at[pl.ds(off, CHUNK)], ia_v.at[b], sia[b])
            pltpu.async_copy(ib.at[pl.ds(off, CHUNK)], ib_v.at[b], sib[b])

        @pl.loop(0, n_chunks, step=2)
        def _(i0):
            for b in range(2):
                o = 1 - b
                off = base + (i0 + b) * CHUNK
                pltpu.make_async_copy(ia.at[pl.ds(off, CHUNK)],
                                      ia_v.at[b], sia[b]).wait()
                pltpu.make_async_copy(ib.at[pl.ds(off, CHUNK)],
                                      ib_v.at[b], sib[b]).wait()

                def _free():
                    pltpu.make_async_copy(ra_v.at[b],
                                          oa.at[pl.ds(off, CHUNK)],
                                          sta[b]).wait()
                    pltpu.make_async_copy(rb_v.at[b],
                                          ob.at[pl.ds(off, CHUNK)],
                                          stb[b]).wait()
                pl.when(i0 > 0)(_free)

                ca = pltpu.async_copy(ta.at[ia_v.at[b]], ra_v.at[b], sga[b])
                cb = pltpu.async_copy(tb.at[ib_v.at[b]], rb_v.at[b], sgb[b])
                ca.wait()
                cb.wait()

                def _prefetch():
                    noff = off + 2 * CHUNK
                    pltpu.async_copy(ia.at[pl.ds(noff, CHUNK)],
                                     ia_v.at[b], sia[b])
                    pltpu.async_copy(ib.at[pl.ds(noff, CHUNK)],
                                     ib_v.at[b], sib[b])
                pl.when(i0 < n_chunks - 2)(_prefetch)

                pltpu.async_copy(ra_v.at[b], oa.at[pl.ds(off, CHUNK)],
                                 sta[b])
                pltpu.async_copy(rb_v.at[b], ob.at[pl.ds(off, CHUNK)],
                                 stb[b])

        for b in range(2):
            off = base + (n_chunks - 2 + b) * CHUNK
            pltpu.make_async_copy(ra_v.at[b], oa.at[pl.ds(off, CHUNK)],
                                  sta[b]).wait()
            pltpu.make_async_copy(rb_v.at[b], ob.at[pl.ds(off, CHUNK)],
                                  stb[b]).wait()

    return k(table_a, table_b, idx_a, idx_b)


def _scatter_body(m_slice_fn, d_hbm, out, m_v, d_v, acc, sems,
                  z_hbm, base, n_chunks, c, s):
    sdm, smm, ssc = sems[0:2], sems[2:4], sems[4:6]

    for j in range(RT // ZR):
        pltpu.sync_copy(z_hbm, acc.at[pl.ds(s * RT + j * ZR, ZR)])
    plsc.subcore_barrier()

    for b in range(2):
        off = base + b * CHUNK
        pltpu.async_copy(d_hbm.at[pl.ds(off, CHUNK)], d_v.at[b], sdm[b])
        pltpu.async_copy(m_slice_fn(off), m_v.at[b], smm[b])

    @pl.loop(0, n_chunks, step=2)
    def _(i0):
        for b in range(2):
            o = 1 - b
            off = base + (i0 + b) * CHUNK
            pltpu.make_async_copy(d_hbm.at[pl.ds(off, CHUNK)],
                                  d_v.at[b], sdm[b]).wait()
            pltpu.make_async_copy(m_slice_fn(off), m_v.at[b], smm[b]).wait()
            pltpu.async_copy(m_v.at[b], acc.at[d_v.at[b]], ssc[b], add=True)

            def _finish():
                pltpu.make_async_copy(m_v.at[o], acc.at[d_v.at[o]],
                                      ssc[o]).wait()
                def _prefetch():
                    noff = off + CHUNK
                    pltpu.async_copy(d_hbm.at[pl.ds(noff, CHUNK)],
                                     d_v.at[o], sdm[o])
                    pltpu.async_copy(m_slice_fn(noff), m_v.at[o], smm[o])
                if b == 0:
                    _prefetch()
                else:
                    pl.when(i0 < n_chunks - 2)(_prefetch)
            if b == 1:
                _finish()
            else:
                pl.when(i0 > 0)(_finish)

    pltpu.make_async_copy(m_v.at[1], acc.at[d_v.at[1]], ssc[1]).wait()
    plsc.subcore_barrier()
    pltpu.sync_copy(acc.at[pl.ds(s * RT, RT)],
                    out.at[pl.ds(c * NP + s * RT, RT)])


def _scatter_scratch():
    return [
        pltpu.VMEM((2, CHUNK, D_), F32),
        pltpu.VMEM((2, CHUNK), I32),
        pltpu.VMEM_SHARED((NP, D_), F32),
    ] + [pltpu.SemaphoreType.DMA] * 6


def _sc_scatter_headsplit(m2, dst, zeros):
    @functools.partial(
        pl.kernel,
        out_type=jax.ShapeDtypeStruct((2 * NP, D_), F32),
        mesh=_mesh(),
        scratch_types=_scatter_scratch(),
    )
    def k(m_hbm, d_hbm, z_hbm, out, m_v, d_v, acc, *sems):
        c = lax.axis_index("c")
        s = lax.axis_index("s")
        base = s * EW2
        _scatter_body(lambda off: m_hbm.at[c, pl.ds(off, CHUNK)],
                      d_hbm, out, m_v, d_v, acc, sems,
                      z_hbm, base, EC2, c, s)

    return k(m2, dst, zeros)


def _sc_scatter_half(m, dst, zeros):
    @functools.partial(
        pl.kernel,
        out_type=jax.ShapeDtypeStruct((2 * NP, D_), F32),
        mesh=_mesh(),
        scratch_types=_scatter_scratch(),
    )
    def k(m_hbm, d_hbm, z_hbm, out, m_v, d_v, acc, *sems):
        c = lax.axis_index("c")
        s = lax.axis_index("s")
        base = (s * NC + c) * EW
        _scatter_body(lambda off: m_hbm.at[pl.ds(off, CHUNK)],
                      d_hbm, out, m_v, d_v, acc, sems,
                      z_hbm, base, EC, c, s)

    return k(m, dst, zeros)


def _tc_mm2(x, wa, wb):
    n, din = x.shape
    dout = wa.shape[1]
    bn = 1000

    def body(x_ref, wa_ref, wb_ref, oa_ref, ob_ref):
        xv = x_ref[...]
        oa_ref[...] = jnp.dot(xv, wa_ref[...], preferred_element_type=F32)
        ob_ref[...] = jnp.dot(xv, wb_ref[...], preferred_element_type=F32)

    return pl.pallas_call(
        body,
        grid=(n // bn,),
        in_specs=[pl.BlockSpec((bn, din), lambda i: (i, 0)),
                  pl.BlockSpec((din, dout), lambda i: (0, 0)),
                  pl.BlockSpec((din, dout), lambda i: (0, 0))],
        out_specs=(pl.BlockSpec((bn, dout), lambda i: (i, 0)),
                   pl.BlockSpec((bn, dout), lambda i: (i, 0))),
        out_shape=(jax.ShapeDtypeStruct((n, dout), F32),
                   jax.ShapeDtypeStruct((n, dout), F32)),
    )(x, wa, wb)


def _tc_score1(a, b, att_flat):
    be = 2048

    def body(a_ref, b_ref, att_ref, m_ref):
        i = pl.program_id(0)
        av = a_ref[...]
        e = av + b_ref[...]
        e = jnp.maximum(e, 0.2 * e)
        ea = e * att_ref[...]
        s0 = jnp.sum(ea[:, :HD_], axis=1, keepdims=True)
        s1 = jnp.sum(ea[:, HD_:], axis=1, keepdims=True)
        rid = i * be + lax.broadcasted_iota(I32, (be, 1), 0)
        mask = (rid < E1).astype(F32)
        x0 = jnp.exp(s0) * mask
        x1 = jnp.exp(s1) * mask
        z = jnp.zeros((be, D_ - HD_ - 1), F32)
        m_ref[0] = jnp.concatenate([av[:, :HD_] * x0, x0, z], axis=1)
        m_ref[1] = jnp.concatenate([av[:, HD_:] * x1, x1, z], axis=1)

    return pl.pallas_call(
        body,
        grid=(EP // be,),
        in_specs=[pl.BlockSpec((be, 2 * HD_), lambda i: (i, 0)),
                  pl.BlockSpec((be, 2 * HD_), lambda i: (i, 0)),
                  pl.BlockSpec((1, 2 * HD_), lambda i: (0, 0))],
        out_specs=pl.BlockSpec((2, be, D_), lambda i: (0, i, 0)),
        out_shape=jax.ShapeDtypeStruct((2, EP, D_), F32),
    )(a, b, att_flat)


def _tc_score2(a, b, att_flat):
    be = 2048

    def body(a_ref, b_ref, att_ref, m_ref):
        i = pl.program_id(0)
        av = a_ref[:, :HD_]
        e = av + b_ref[:, HD_:]
        e = jnp.maximum(e, 0.2 * e)
        s0 = jnp.sum(e * att_ref[...], axis=1, keepdims=True)
        rid = i * be + lax.broadcasted_iota(I32, (be, 1), 0)
        mask = (rid < E1).astype(F32)
        x0 = jnp.exp(s0) * mask
        m_ref[...] = jnp.concatenate(
            [av * x0, x0, jnp.zeros((be, D_ - HD_ - 1), F32)], axis=1)

    return pl.pallas_call(
        body,
        grid=(EP // be,),
        in_specs=[pl.BlockSpec((be, D_), lambda i: (i, 0)),
                  pl.BlockSpec((be, D_), lambda i: (i, 0)),
                  pl.BlockSpec((1, HD_), lambda i: (0, 0))],
        out_specs=pl.BlockSpec((be, D_), lambda i: (i, 0)),
        out_shape=jax.ShapeDtypeStruct((EP, D_), F32),
    )(a, b, att_flat)


def _tc_combine1(p0, p1, b1, wl, wr):
    bn = 1000

    def body(p0_ref, p1_ref, b1_ref, wl_ref, wr_ref, t_ref):
        pa = p0_ref[...]
        pb = p1_ref[...]
        h0 = pa[:, :HD_] / (pa[:, HD_:HD_ + 1] + 1e-16)
        h1 = pb[:, :HD_] / (pb[:, HD_:HD_ + 1] + 1e-16)
        h = jnp.maximum(jnp.concatenate([h0, h1], axis=1) + b1_ref[...], 0.0)
        t_ref[:, :HD_] = jnp.dot(h, wl_ref[...], preferred_element_type=F32)
        t_ref[:, HD_:] = jnp.dot(h, wr_ref[...], preferred_element_type=F32)

    return pl.pallas_call(
        body,
        grid=(N_ // bn,),
        in_specs=[pl.BlockSpec((bn, D_), lambda i: (i, 0)),
                  pl.BlockSpec((bn, D_), lambda i: (i, 0)),
                  pl.BlockSpec((1, 2 * HD_), lambda i: (0, 0)),
                  pl.BlockSpec((2 * HD_, HD_), lambda i: (0, 0)),
                  pl.BlockSpec((2 * HD_, HD_), lambda i: (0, 0))],
        out_specs=pl.BlockSpec((bn, D_), lambda i: (i, 0)),
        out_shape=jax.ShapeDtypeStruct((N_, D_), F32),
    )(p0, p1, b1, wl, wr)


def _tc_combine2(q0, q1, b2):
    bn = 1000

    def body(q0_ref, q1_ref, b2_ref, h_ref, hp_ref):
        acc = q0_ref[...] + q1_ref[...]
        hv = acc[:, :HD_] / (acc[:, HD_:HD_ + 1] + 1e-16)
        hv = jnp.maximum(hv + b2_ref[...], 0.0)
        h_ref[...] = hv
        hp_ref[...] = jnp.concatenate([hv, jnp.zeros((bn, D_ - HD_), F32)],
                                      axis=1)

    return pl.pallas_call(
        body,
        grid=(N_ // bn,),
        in_specs=[pl.BlockSpec((bn, D_), lambda i: (i, 0)),
                  pl.BlockSpec((bn, D_), lambda i: (i, 0)),
                  pl.BlockSpec((1, HD_), lambda i: (0, 0))],
        out_specs=(pl.BlockSpec((bn, HD_), lambda i: (i, 0)),
                   pl.BlockSpec((bn, D_), lambda i: (i, 0))),
        out_shape=(jax.ShapeDtypeStruct((N_, HD_), F32),
                   jax.ShapeDtypeStruct((N_, D_), F32)),
    )(q0, q1, b2)


def _tc_edge_mlp(hs, hd, ea, w3a, w3b, w3c, b3, w4, b4):
    bp = 2048

    def body(hs_ref, hd_ref, ea_ref, w3a_ref, w3b_ref, w3c_ref, b3_ref,
             w4_ref, b4_ref, o_ref):
        hid = (jnp.dot(hs_ref[...], w3a_ref[...], preferred_element_type=F32)
               + jnp.dot(hd_ref[...], w3b_ref[...], preferred_element_type=F32)
               + ea_ref[:, 0:1] * w3c_ref[0:1, :]
               + ea_ref[:, 1:2] * w3c_ref[1:2, :]
               + b3_ref[...])
        hid = jnp.maximum(hid, 0.0)
        o_ref[...] = jnp.dot(hid, w4_ref[...],
                             preferred_element_type=F32) + b4_ref[...]

    return pl.pallas_call(
        body,
        grid=(PP // bp,),
        in_specs=[pl.BlockSpec((bp, D_), lambda i: (i, 0)),
                  pl.BlockSpec((bp, D_), lambda i: (i, 0)),
                  pl.BlockSpec((bp, 2), lambda i: (i, 0)),
                  pl.BlockSpec((D_, HD_), lambda i: (0, 0)),
                  pl.BlockSpec((D_, HD_), lambda i: (0, 0)),
                  pl.BlockSpec((2, HD_), lambda i: (0, 0)),
                  pl.BlockSpec((1, HD_), lambda i: (0, 0)),
                  pl.BlockSpec((HD_, 1), lambda i: (0, 0)),
                  pl.BlockSpec((1, 1), lambda i: (0, 0))],
        out_specs=pl.BlockSpec((bp, 1), lambda i: (i, 0)),
        out_shape=jax.ShapeDtypeStruct((PP, 1), F32),
    )(hs, hd, ea, w3a, w3b, w3c, b3, w4, b4)


@jax.jit
def kernel(x, edge_index, edge_pairs, edge_attr, Wl1, Wr1, att1, b1,
           Wl2, Wr2, att2, b2, W3, b3, W4, b4):
    ei = edge_index.astype(I32)
    loop_idx = jnp.arange(N_, dtype=I32)
    epad = jnp.zeros((EP - E1,), I32)
    srcp = jnp.concatenate([ei[0], loop_idx, epad])
    dstp = jnp.concatenate([ei[1], loop_idx, epad])
    zeros = jnp.zeros((ZR, D_), F32)

    xl, xr = _tc_mm2(x, Wl1, Wr1)
    xl2c = jnp.tile(xl[None], (2, 1, 1))
    xr2c = jnp.tile(xr[None], (2, 1, 1))
    a1, b1g = _sc_gather2(xl2c, xr2c, srcp, dstp, EW, EC, EP)
    m1 = _tc_score1(a1, b1g, att1.reshape(1, 2 * HD_))
    parts1 = _sc_scatter_headsplit(m1, dstp, zeros)
    t2 = _tc_combine1(parts1[:N_], parts1[NP:NP + N_],
                      b1.reshape(1, 2 * HD_), Wl2, Wr2)

    t2q = jnp.tile(t2[None], (4, 1, 1))
    a2, b2g = _sc_gather2(t2q[:2], t2q[2:], srcp, dstp, EW, EC, EP)
    m2 = _tc_score2(a2, b2g, att2.reshape(1, HD_))
    parts2 = _sc_scatter_half(m2, dstp, zeros)
    h, hp = _tc_combine2(parts2[:N_], parts2[NP:NP + N_], b2.reshape(1, HD_))

    ep = edge_pairs.astype(I32)
    ppad = jnp.zeros((PP - P_,), I32)
    spp = jnp.concatenate([ep[0], ppad])
    dpp = jnp.concatenate([ep[1], ppad])
    eap = jnp.concatenate([edge_attr, jnp.zeros((PP - P_, 2), F32)], axis=0)
    hq = jnp.tile(hp[None], (4, 1, 1))
    hs, hdg = _sc_gather2(hq[:2], hq[2:], spp, dpp, PW, PC, PP)
    zw = jnp.zeros((HD_, HD_), F32)
    w3a = jnp.concatenate([W3[:HD_], zw], axis=0)
    w3b = jnp.concatenate([W3[HD_:2 * HD_], zw], axis=0)
    out = _tc_edge_mlp(hs, hdg, eap, w3a, w3b, W3[2 * HD_:],
                       b3.reshape(1, HD_), W4, b4.reshape(1, 1))
    return (out[:P_, 0], h)


# speedup vs baseline: 1.9773x; 1.9773x over previous
import functools

import jax
import jax.numpy as jnp
from jax import lax
from jax.experimental import pallas as pl
from jax.experimental.pallas import tpu as pltpu
from jax.experimental.pallas import tpu_sc as plsc

F32 = jnp.float32
I32 = jnp.int32

N_ = 10000
E_ = 320000
P_ = 200000
HD_ = 64
E1 = E_ + N_

NC, NS = 2, 16
NW = NC * NS
CHUNK = 128
D_ = 128

EC = 82
EW = CHUNK * EC
EP = NW * EW
EC2 = 2 * EC
EW2 = CHUNK * EC2

PC = 50
PW = CHUNK * PC
PP = NW * PW

NP = 10240
RT = NP // NS
ZR = 128


def _mesh():
    return plsc.VectorSubcoreMesh(core_axis_name="c", subcore_axis_name="s")


def _sc_gather_stk(tstk, istk, total):
    per_tile = total // NS
    n_chunks = per_tile // CHUNK

    @functools.partial(
        pl.kernel,
        out_type=jax.ShapeDtypeStruct((2, total, D_), F32),
        mesh=_mesh(),
        scratch_types=[
            pltpu.VMEM((2, CHUNK), I32),
            pltpu.VMEM((2, CHUNK, D_), F32),
            pltpu.VMEM_SHARED((NP, D_), F32),
        ] + [pltpu.SemaphoreType.DMA] * 6,
    )
    def k(t_hbm, i_hbm, out, i_v, r_v, tab, *sems):
        si, sg, st = sems[0:2], sems[2:4], sems[4:6]
        c = lax.axis_index("c")
        s = lax.axis_index("s")
        pltpu.sync_copy(t_hbm.at[c, pl.ds(s * RT, RT)],
                        tab.at[pl.ds(s * RT, RT)])
        plsc.subcore_barrier()

        base = s * per_tile
        for b in range(2):
            off = base + b * CHUNK
            pltpu.async_copy(i_hbm.at[c, pl.ds(off, CHUNK)], i_v.at[b], si[b])

        @pl.loop(0, n_chunks, step=2)
        def _(i0):
            for b in range(2):
                off = base + (i0 + b) * CHUNK
                pltpu.make_async_copy(i_hbm.at[c, pl.ds(off, CHUNK)],
                                      i_v.at[b], si[b]).wait()

                def _free():
                    pltpu.make_async_copy(r_v.at[b],
                                          out.at[c, pl.ds(off, CHUNK)],
                                          st[b]).wait()
                pl.when(i0 > 0)(_free)

                pltpu.async_copy(tab.at[i_v.at[b]], r_v.at[b], sg[b]).wait()

                def _prefetch():
                    noff = off + 2 * CHUNK
                    pltpu.async_copy(i_hbm.at[c, pl.ds(noff, CHUNK)],
                                     i_v.at[b], si[b])
                pl.when(i0 < n_chunks - 2)(_prefetch)

                pltpu.async_copy(r_v.at[b], out.at[c, pl.ds(off, CHUNK)],
                                 st[b])

        for b in range(2):
            off = base + (n_chunks - 2 + b) * CHUNK
            pltpu.make_async_copy(r_v.at[b], out.at[c, pl.ds(off, CHUNK)],
                                  st[b]).wait()

    return k(tstk, istk)


def _scatter_body(m_slice_fn, d_hbm, out, m_v, d_v, acc, sems,
                  z_hbm, base, n_chunks, c, s):
    sdm, smm, ssc = sems[0:2], sems[2:4], sems[4:6]

    for j in range(RT // ZR):
        pltpu.sync_copy(z_hbm, acc.at[pl.ds(s * RT + j * ZR, ZR)])
    plsc.subcore_barrier()

    for b in range(2):
        off = base + b * CHUNK
        pltpu.async_copy(d_hbm.at[pl.ds(off, CHUNK)], d_v.at[b], sdm[b])
        pltpu.async_copy(m_slice_fn(off), m_v.at[b], smm[b])

    @pl.loop(0, n_chunks, step=2)
    def _(i0):
        for b in range(2):
            o = 1 - b
            off = base + (i0 + b) * CHUNK
            pltpu.make_async_copy(d_hbm.at[pl.ds(off, CHUNK)],
                                  d_v.at[b], sdm[b]).wait()
            pltpu.make_async_copy(m_slice_fn(off), m_v.at[b], smm[b]).wait()
            pltpu.async_copy(m_v.at[b], acc.at[d_v.at[b]], ssc[b], add=True)

            def _finish():
                pltpu.make_async_copy(m_v.at[o], acc.at[d_v.at[o]],
                                      ssc[o]).wait()
                def _prefetch():
                    noff = off + CHUNK
                    pltpu.async_copy(d_hbm.at[pl.ds(noff, CHUNK)],
                                     d_v.at[o], sdm[o])
                    pltpu.async_copy(m_slice_fn(noff), m_v.at[o], smm[o])
                if b == 0:
                    _prefetch()
                else:
                    pl.when(i0 < n_chunks - 2)(_prefetch)
            if b == 1:
                _finish()
            else:
                pl.when(i0 > 0)(_finish)

    pltpu.make_async_copy(m_v.at[1], acc.at[d_v.at[1]], ssc[1]).wait()
    plsc.subcore_barrier()
    pltpu.sync_copy(acc.at[pl.ds(s * RT, RT)],
                    out.at[pl.ds(c * NP + s * RT, RT)])


def _scatter_scratch():
    return [
        pltpu.VMEM((2, CHUNK, D_), F32),
        pltpu.VMEM((2, CHUNK), I32),
        pltpu.VMEM_SHARED((NP, D_), F32),
    ] + [pltpu.SemaphoreType.DMA] * 6


def _sc_scatter_headsplit(m2, dst, zeros):
    @functools.partial(
        pl.kernel,
        out_type=jax.ShapeDtypeStruct((2 * NP, D_), F32),
        mesh=_mesh(),
        scratch_types=_scatter_scratch(),
    )
    def k(m_hbm, d_hbm, z_hbm, out, m_v, d_v, acc, *sems):
        c = lax.axis_index("c")
        s = lax.axis_index("s")
        base = s * EW2
        _scatter_body(lambda off: m_hbm.at[c, pl.ds(off, CHUNK)],
                      d_hbm, out, m_v, d_v, acc, sems,
                      z_hbm, base, EC2, c, s)

    return k(m2, dst, zeros)


def _sc_scatter_half(m, dst, zeros):
    @functools.partial(
        pl.kernel,
        out_type=jax.ShapeDtypeStruct((2 * NP, D_), F32),
        mesh=_mesh(),
        scratch_types=_scatter_scratch(),
    )
    def k(m_hbm, d_hbm, z_hbm, out, m_v, d_v, acc, *sems):
        c = lax.axis_index("c")
        s = lax.axis_index("s")
        base = (s * NC + c) * EW
        _scatter_body(lambda off: m_hbm.at[pl.ds(off, CHUNK)],
                      d_hbm, out, m_v, d_v, acc, sems,
                      z_hbm, base, EC, c, s)

    return k(m, dst, zeros)


def _tc_mm2(x, wa, wb):
    n, din = x.shape
    dout = wa.shape[1]
    bn = 1000

    def body(x_ref, wa_ref, wb_ref, oa_ref, ob_ref):
        xv = x_ref[...]
        oa_ref[...] = jnp.dot(xv, wa_ref[...], preferred_element_type=F32)
        ob_ref[...] = jnp.dot(xv, wb_ref[...], preferred_element_type=F32)

    return pl.pallas_call(
        body,
        grid=(n // bn,),
        in_specs=[pl.BlockSpec((bn, din), lambda i: (i, 0)),
                  pl.BlockSpec((din, dout), lambda i: (0, 0)),
                  pl.BlockSpec((din, dout), lambda i: (0, 0))],
        out_specs=(pl.BlockSpec((bn, dout), lambda i: (i, 0)),
                   pl.BlockSpec((bn, dout), lambda i: (i, 0))),
        out_shape=(jax.ShapeDtypeStruct((n, dout), F32),
                   jax.ShapeDtypeStruct((n, dout), F32)),
    )(x, wa, wb)


def _tc_score1(a, b, att_flat):
    be = 2048

    def body(a_ref, b_ref, att_ref, m_ref):
        i = pl.program_id(0)
        av = a_ref[0]
        e = av + b_ref[0]
        e = jnp.maximum(e, 0.2 * e)
        ea = e * att_ref[...]
        s0 = jnp.sum(ea[:, :HD_], axis=1, keepdims=True)
        s1 = jnp.sum(ea[:, HD_:], axis=1, keepdims=True)
        rid = i * be + lax.broadcasted_iota(I32, (be, 1), 0)
        mask = (rid < E1).astype(F32)
        x0 = jnp.exp(s0) * mask
        x1 = jnp.exp(s1) * mask
        z = jnp.zeros((be, D_ - HD_ - 1), F32)
        m_ref[0] = jnp.concatenate([av[:, :HD_] * x0, x0, z], axis=1)
        m_ref[1] = jnp.concatenate([av[:, HD_:] * x1, x1, z], axis=1)

    return pl.pallas_call(
        body,
        grid=(EP // be,),
        in_specs=[pl.BlockSpec((1, be, D_), lambda i: (0, i, 0)),
                  pl.BlockSpec((1, be, D_), lambda i: (1, i, 0)),
                  pl.BlockSpec((1, 2 * HD_), lambda i: (0, 0))],
        out_specs=pl.BlockSpec((2, be, D_), lambda i: (0, i, 0)),
        out_shape=jax.ShapeDtypeStruct((2, EP, D_), F32),
    )(a, b, att_flat)


def _tc_score2(a, b, att_flat):
    be = 2048

    def body(a_ref, b_ref, att_ref, m_ref):
        i = pl.program_id(0)
        av = a_ref[0][:, :HD_]
        e = av + b_ref[0][:, HD_:]
        e = jnp.maximum(e, 0.2 * e)
        s0 = jnp.sum(e * att_ref[...], axis=1, keepdims=True)
        rid = i * be + lax.broadcasted_iota(I32, (be, 1), 0)
        mask = (rid < E1).astype(F32)
        x0 = jnp.exp(s0) * mask
        m_ref[...] = jnp.concatenate(
            [av * x0, x0, jnp.zeros((be, D_ - HD_ - 1), F32)], axis=1)

    return pl.pallas_call(
        body,
        grid=(EP // be,),
        in_specs=[pl.BlockSpec((1, be, D_), lambda i: (0, i, 0)),
                  pl.BlockSpec((1, be, D_), lambda i: (1, i, 0)),
                  pl.BlockSpec((1, HD_), lambda i: (0, 0))],
        out_specs=pl.BlockSpec((be, D_), lambda i: (i, 0)),
        out_shape=jax.ShapeDtypeStruct((EP, D_), F32),
    )(a, b, att_flat)


def _tc_combine1(p0, p1, b1, wl, wr):
    bn = 1000

    def body(p0_ref, p1_ref, b1_ref, wl_ref, wr_ref, t_ref):
        pa = p0_ref[...]
        pb = p1_ref[...]
        h0 = pa[:, :HD_] / (pa[:, HD_:HD_ + 1] + 1e-16)
        h1 = pb[:, :HD_] / (pb[:, HD_:HD_ + 1] + 1e-16)
        h = jnp.maximum(jnp.concatenate([h0, h1], axis=1) + b1_ref[...], 0.0)
        t_ref[:, :HD_] = jnp.dot(h, wl_ref[...], preferred_element_type=F32)
        t_ref[:, HD_:] = jnp.dot(h, wr_ref[...], preferred_element_type=F32)

    return pl.pallas_call(
        body,
        grid=(N_ // bn,),
        in_specs=[pl.BlockSpec((bn, D_), lambda i: (i, 0)),
                  pl.BlockSpec((bn, D_), lambda i: (i, 0)),
                  pl.BlockSpec((1, 2 * HD_), lambda i: (0, 0)),
                  pl.BlockSpec((2 * HD_, HD_), lambda i: (0, 0)),
                  pl.BlockSpec((2 * HD_, HD_), lambda i: (0, 0))],
        out_specs=pl.BlockSpec((bn, D_), lambda i: (i, 0)),
        out_shape=jax.ShapeDtypeStruct((N_, D_), F32),
    )(p0, p1, b1, wl, wr)


def _tc_combine2(q0, q1, b2):
    bn = 1000

    def body(q0_ref, q1_ref, b2_ref, h_ref, hp_ref):
        acc = q0_ref[...] + q1_ref[...]
        hv = acc[:, :HD_] / (acc[:, HD_:HD_ + 1] + 1e-16)
        hv = jnp.maximum(hv + b2_ref[...], 0.0)
        h_ref[...] = hv
        hp_ref[...] = jnp.concatenate([hv, jnp.zeros((bn, D_ - HD_), F32)],
                                      axis=1)

    return pl.pallas_call(
        body,
        grid=(N_ // bn,),
        in_specs=[pl.BlockSpec((bn, D_), lambda i: (i, 0)),
                  pl.BlockSpec((bn, D_), lambda i: (i, 0)),
                  pl.BlockSpec((1, HD_), lambda i: (0, 0))],
        out_specs=(pl.BlockSpec((bn, HD_), lambda i: (i, 0)),
                   pl.BlockSpec((bn, D_), lambda i: (i, 0))),
        out_shape=(jax.ShapeDtypeStruct((N_, HD_), F32),
                   jax.ShapeDtypeStruct((N_, D_), F32)),
    )(q0, q1, b2)


def _tc_edge_mlp(hs, hd, ea, w3a, w3b, w3c, b3, w4, b4):
    bp = 2048

    def body(hs_ref, hd_ref, ea_ref, w3a_ref, w3b_ref, w3c_ref, b3_ref,
             w4_ref, b4_ref, o_ref):
        hid = (jnp.dot(hs_ref[0], w3a_ref[...], preferred_element_type=F32)
               + jnp.dot(hd_ref[0], w3b_ref[...], preferred_element_type=F32)
               + ea_ref[:, 0:1] * w3c_ref[0:1, :]
               + ea_ref[:, 1:2] * w3c_ref[1:2, :]
               + b3_ref[...])
        hid = jnp.maximum(hid, 0.0)
        o_ref[...] = jnp.dot(hid, w4_ref[...],
                             preferred_element_type=F32) + b4_ref[...]

    return pl.pallas_call(
        body,
        grid=(PP // bp,),
        in_specs=[pl.BlockSpec((1, bp, D_), lambda i: (0, i, 0)),
                  pl.BlockSpec((1, bp, D_), lambda i: (1, i, 0)),
                  pl.BlockSpec((bp, 2), lambda i: (i, 0)),
                  pl.BlockSpec((D_, HD_), lambda i: (0, 0)),
                  pl.BlockSpec((D_, HD_), lambda i: (0, 0)),
                  pl.BlockSpec((2, HD_), lambda i: (0, 0)),
                  pl.BlockSpec((1, HD_), lambda i: (0, 0)),
                  pl.BlockSpec((HD_, 1), lambda i: (0, 0)),
                  pl.BlockSpec((1, 1), lambda i: (0, 0))],
        out_specs=pl.BlockSpec((bp, 1), lambda i: (i, 0)),
        out_shape=jax.ShapeDtypeStruct((PP, 1), F32),
    )(hs, hd, ea, w3a, w3b, w3c, b3, w4, b4)


@jax.jit
def kernel(x, edge_index, edge_pairs, edge_attr, Wl1, Wr1, att1, b1,
           Wl2, Wr2, att2, b2, W3, b3, W4, b4):
    ei = edge_index.astype(I32)
    loop_idx = jnp.arange(N_, dtype=I32)
    epad = jnp.zeros((EP - E1,), I32)
    srcp = jnp.concatenate([ei[0], loop_idx, epad])
    dstp = jnp.concatenate([ei[1], loop_idx, epad])
    zeros = jnp.zeros((ZR, D_), F32)

    istk = jnp.stack([srcp, dstp])
    padrows = jnp.zeros((NP - N_, D_), F32)

    xl, xr = _tc_mm2(x, Wl1, Wr1)
    t1 = jnp.stack([jnp.concatenate([xl, padrows]),
                    jnp.concatenate([xr, padrows])])
    ab1 = _sc_gather_stk(t1, istk, EP)
    m1 = _tc_score1(ab1, ab1, att1.reshape(1, 2 * HD_))
    parts1 = _sc_scatter_headsplit(m1, dstp, zeros)
    t2 = _tc_combine1(parts1[:N_], parts1[NP:NP + N_],
                      b1.reshape(1, 2 * HD_), Wl2, Wr2)

    t2p = jnp.concatenate([t2, padrows])
    ab2 = _sc_gather_stk(jnp.stack([t2p, t2p]), istk, EP)
    m2 = _tc_score2(ab2, ab2, att2.reshape(1, HD_))
    parts2 = _sc_scatter_half(m2, dstp, zeros)
    h, hp = _tc_combine2(parts2[:N_], parts2[NP:NP + N_], b2.reshape(1, HD_))

    ep = edge_pairs.astype(I32)
    ppad = jnp.zeros((PP - P_,), I32)
    spp = jnp.concatenate([ep[0], ppad])
    dpp = jnp.concatenate([ep[1], ppad])
    eap = jnp.concatenate([edge_attr, jnp.zeros((PP - P_, 2), F32)], axis=0)
    hpp = jnp.concatenate([hp, padrows])
    hsd = _sc_gather_stk(jnp.stack([hpp, hpp]), jnp.stack([spp, dpp]), PP)
    hs = hdg = hsd
    zw = jnp.zeros((HD_, HD_), F32)
    w3a = jnp.concatenate([W3[:HD_], zw], axis=0)
    w3b = jnp.concatenate([W3[HD_:2 * HD_], zw], axis=0)
    out = _tc_edge_mlp(hs, hdg, eap, w3a, w3b, W3[2 * HD_:],
                       b3.reshape(1, HD_), W4, b4.reshape(1, 1))
    return (out[:P_, 0], h)


# speedup vs baseline: 1.9863x; 1.0045x over previous
import functools

import jax
import jax.numpy as jnp
from jax import lax
from jax.experimental import pallas as pl
from jax.experimental.pallas import tpu as pltpu
from jax.experimental.pallas import tpu_sc as plsc

F32 = jnp.float32
I32 = jnp.int32

N_ = 10000
E_ = 320000
P_ = 200000
HD_ = 64
E1 = E_ + N_

NC, NS = 2, 16
NW = NC * NS
CHUNK = 128
D_ = 128

EC = 82
EW = CHUNK * EC
EP = NW * EW
EC2 = 2 * EC
EW2 = CHUNK * EC2

PC = 50
PW = CHUNK * PC
PP = NW * PW

NP = 10240
RT = NP // NS
ZR = 128


def _mesh():
    return plsc.VectorSubcoreMesh(core_axis_name="c", subcore_axis_name="s")


def _sc_gather_stk(tstk, istk, total):
    per_tile = total // NS
    n_chunks = per_tile // CHUNK

    @functools.partial(
        pl.kernel,
        out_type=jax.ShapeDtypeStruct((2, total, D_), F32),
        mesh=_mesh(),
        scratch_types=[
            pltpu.VMEM((2, CHUNK), I32),
            pltpu.VMEM((2, CHUNK, D_), F32),
            pltpu.VMEM_SHARED((NP, D_), F32),
        ] + [pltpu.SemaphoreType.DMA] * 6,
    )
    def k(t_hbm, i_hbm, out, i_v, r_v, tab, *sems):
        si, sg, st = sems[0:2], sems[2:4], sems[4:6]
        c = lax.axis_index("c")
        s = lax.axis_index("s")
        pltpu.sync_copy(t_hbm.at[c, pl.ds(s * RT, RT)],
                        tab.at[pl.ds(s * RT, RT)])
        plsc.subcore_barrier()

        base = s * per_tile
        for b in range(2):
            off = base + b * CHUNK
            pltpu.async_copy(i_hbm.at[c, pl.ds(off, CHUNK)], i_v.at[b], si[b])

        @pl.loop(0, n_chunks, step=2)
        def _(i0):
            for b in range(2):
                o = 1 - b
                off = base + (i0 + b) * CHUNK
                pltpu.make_async_copy(i_hbm.at[c, pl.ds(off, CHUNK)],
                                      i_v.at[b], si[b]).wait()

                def _free():
                    pltpu.make_async_copy(r_v.at[b],
                                          out.at[c, pl.ds(off, CHUNK)],
                                          st[b]).wait()
                pl.when(i0 > 0)(_free)

                pltpu.async_copy(tab.at[i_v.at[b]], r_v.at[b], sg[b])

                def _finish():
                    poff = off - CHUNK
                    pltpu.make_async_copy(tab.at[i_v.at[o]], r_v.at[o],
                                          sg[o]).wait()

                    def _prefetch():
                        noff = off + CHUNK
                        pltpu.async_copy(i_hbm.at[c, pl.ds(noff, CHUNK)],
                                         i_v.at[o], si[o])
                    if b == 0:
                        _prefetch()
                    else:
                        pl.when(i0 < n_chunks - 2)(_prefetch)
                    pltpu.async_copy(r_v.at[o], out.at[c, pl.ds(poff, CHUNK)],
                                     st[o])
                if b == 1:
                    _finish()
                else:
                    pl.when(i0 > 0)(_finish)

        loff = base + (n_chunks - 1) * CHUNK
        pltpu.make_async_copy(tab.at[i_v.at[1]], r_v.at[1], sg[1]).wait()
        pltpu.async_copy(r_v.at[1], out.at[c, pl.ds(loff, CHUNK)], st[1])
        for b in range(2):
            off = base + (n_chunks - 2 + b) * CHUNK
            pltpu.make_async_copy(r_v.at[b], out.at[c, pl.ds(off, CHUNK)],
                                  st[b]).wait()

    return k(tstk, istk)


def _scatter_body(m_slice_fn, d_hbm, out, m_v, d_v, acc, sems,
                  z_hbm, base, n_chunks, c, s):
    sdm, smm, ssc = sems[0:2], sems[2:4], sems[4:6]

    for j in range(RT // ZR):
        pltpu.sync_copy(z_hbm, acc.at[pl.ds(s * RT + j * ZR, ZR)])
    plsc.subcore_barrier()

    for b in range(2):
        off = base + b * CHUNK
        pltpu.async_copy(d_hbm.at[pl.ds(off, CHUNK)], d_v.at[b], sdm[b])
        pltpu.async_copy(m_slice_fn(off), m_v.at[b], smm[b])

    @pl.loop(0, n_chunks, step=2)
    def _(i0):
        for b in range(2):
            o = 1 - b
            off = base + (i0 + b) * CHUNK
            pltpu.make_async_copy(d_hbm.at[pl.ds(off, CHUNK)],
                                  d_v.at[b], sdm[b]).wait()
            pltpu.make_async_copy(m_slice_fn(off), m_v.at[b], smm[b]).wait()
            pltpu.async_copy(m_v.at[b], acc.at[d_v.at[b]], ssc[b], add=True)

            def _finish():
                pltpu.make_async_copy(m_v.at[o], acc.at[d_v.at[o]],
                                      ssc[o]).wait()
                def _prefetch():
                    noff = off + CHUNK
                    pltpu.async_copy(d_hbm.at[pl.ds(noff, CHUNK)],
                                     d_v.at[o], sdm[o])
                    pltpu.async_copy(m_slice_fn(noff), m_v.at[o], smm[o])
                if b == 0:
                    _prefetch()
                else:
                    pl.when(i0 < n_chunks - 2)(_prefetch)
            if b == 1:
                _finish()
            else:
                pl.when(i0 > 0)(_finish)

    pltpu.make_async_copy(m_v.at[1], acc.at[d_v.at[1]], ssc[1]).wait()
    plsc.subcore_barrier()
    pltpu.sync_copy(acc.at[pl.ds(s * RT, RT)],
                    out.at[pl.ds(c * NP + s * RT, RT)])


def _scatter_scratch():
    return [
        pltpu.VMEM((2, CHUNK, D_), F32),
        pltpu.VMEM((2, CHUNK), I32),
        pltpu.VMEM_SHARED((NP, D_), F32),
    ] + [pltpu.SemaphoreType.DMA] * 6


def _sc_scatter_headsplit(m2, dst, zeros):
    @functools.partial(
        pl.kernel,
        out_type=jax.ShapeDtypeStruct((2 * NP, D_), F32),
        mesh=_mesh(),
        scratch_types=_scatter_scratch(),
    )
    def k(m_hbm, d_hbm, z_hbm, out, m_v, d_v, acc, *sems):
        c = lax.axis_index("c")
        s = lax.axis_index("s")
        base = s * EW2
        _scatter_body(lambda off: m_hbm.at[c, pl.ds(off, CHUNK)],
                      d_hbm, out, m_v, d_v, acc, sems,
                      z_hbm, base, EC2, c, s)

    return k(m2, dst, zeros)


def _sc_scatter_half(m, dst, zeros):
    @functools.partial(
        pl.kernel,
        out_type=jax.ShapeDtypeStruct((2 * NP, D_), F32),
        mesh=_mesh(),
        scratch_types=_scatter_scratch(),
    )
    def k(m_hbm, d_hbm, z_hbm, out, m_v, d_v, acc, *sems):
        c = lax.axis_index("c")
        s = lax.axis_index("s")
        base = (s * NC + c) * EW
        _scatter_body(lambda off: m_hbm.at[pl.ds(off, CHUNK)],
                      d_hbm, out, m_v, d_v, acc, sems,
                      z_hbm, base, EC, c, s)

    return k(m, dst, zeros)


def _tc_mm2(x, wa, wb):
    n, din = x.shape
    dout = wa.shape[1]
    bn = 1000

    def body(x_ref, wa_ref, wb_ref, oa_ref, ob_ref):
        xv = x_ref[...]
        oa_ref[...] = jnp.dot(xv, wa_ref[...], preferred_element_type=F32)
        ob_ref[...] = jnp.dot(xv, wb_ref[...], preferred_element_type=F32)

    return pl.pallas_call(
        body,
        grid=(n // bn,),
        in_specs=[pl.BlockSpec((bn, din), lambda i: (i, 0)),
                  pl.BlockSpec((din, dout), lambda i: (0, 0)),
                  pl.BlockSpec((din, dout), lambda i: (0, 0))],
        out_specs=(pl.BlockSpec((bn, dout), lambda i: (i, 0)),
                   pl.BlockSpec((bn, dout), lambda i: (i, 0))),
        out_shape=(jax.ShapeDtypeStruct((n, dout), F32),
                   jax.ShapeDtypeStruct((n, dout), F32)),
    )(x, wa, wb)


def _tc_score1(a, b, att_flat):
    be = 2048

    def body(a_ref, b_ref, att_ref, m_ref):
        i = pl.program_id(0)
        av = a_ref[0]
        e = av + b_ref[0]
        e = jnp.maximum(e, 0.2 * e)
        ea = e * att_ref[...]
        s0 = jnp.sum(ea[:, :HD_], axis=1, keepdims=True)
        s1 = jnp.sum(ea[:, HD_:], axis=1, keepdims=True)
        rid = i * be + lax.broadcasted_iota(I32, (be, 1), 0)
        mask = (rid < E1).astype(F32)
        x0 = jnp.exp(s0) * mask
        x1 = jnp.exp(s1) * mask
        z = jnp.zeros((be, D_ - HD_ - 1), F32)
        m_ref[0] = jnp.concatenate([av[:, :HD_] * x0, x0, z], axis=1)
        m_ref[1] = jnp.concatenate([av[:, HD_:] * x1, x1, z], axis=1)

    return pl.pallas_call(
        body,
        grid=(EP // be,),
        in_specs=[pl.BlockSpec((1, be, D_), lambda i: (0, i, 0)),
                  pl.BlockSpec((1, be, D_), lambda i: (1, i, 0)),
                  pl.BlockSpec((1, 2 * HD_), lambda i: (0, 0))],
        out_specs=pl.BlockSpec((2, be, D_), lambda i: (0, i, 0)),
        out_shape=jax.ShapeDtypeStruct((2, EP, D_), F32),
    )(a, b, att_flat)


def _tc_score2(a, b, att_flat):
    be = 2048

    def body(a_ref, b_ref, att_ref, m_ref):
        i = pl.program_id(0)
        av = a_ref[0][:, :HD_]
        e = av + b_ref[0][:, HD_:]
        e = jnp.maximum(e, 0.2 * e)
        s0 = jnp.sum(e * att_ref[...], axis=1, keepdims=True)
        rid = i * be + lax.broadcasted_iota(I32, (be, 1), 0)
        mask = (rid < E1).astype(F32)
        x0 = jnp.exp(s0) * mask
        m_ref[...] = jnp.concatenate(
            [av * x0, x0, jnp.zeros((be, D_ - HD_ - 1), F32)], axis=1)

    return pl.pallas_call(
        body,
        grid=(EP // be,),
        in_specs=[pl.BlockSpec((1, be, D_), lambda i: (0, i, 0)),
                  pl.BlockSpec((1, be, D_), lambda i: (1, i, 0)),
                  pl.BlockSpec((1, HD_), lambda i: (0, 0))],
        out_specs=pl.BlockSpec((be, D_), lambda i: (i, 0)),
        out_shape=jax.ShapeDtypeStruct((EP, D_), F32),
    )(a, b, att_flat)


def _tc_combine1(p0, p1, b1, wl, wr):
    bn = 1000

    def body(p0_ref, p1_ref, b1_ref, wl_ref, wr_ref, t_ref):
        pa = p0_ref[...]
        pb = p1_ref[...]
        h0 = pa[:, :HD_] / (pa[:, HD_:HD_ + 1] + 1e-16)
        h1 = pb[:, :HD_] / (pb[:, HD_:HD_ + 1] + 1e-16)
        h = jnp.maximum(jnp.concatenate([h0, h1], axis=1) + b1_ref[...], 0.0)
        t_ref[:, :HD_] = jnp.dot(h, wl_ref[...], preferred_element_type=F32)
        t_ref[:, HD_:] = jnp.dot(h, wr_ref[...], preferred_element_type=F32)

    return pl.pallas_call(
        body,
        grid=(N_ // bn,),
        in_specs=[pl.BlockSpec((bn, D_), lambda i: (i, 0)),
                  pl.BlockSpec((bn, D_), lambda i: (i, 0)),
                  pl.BlockSpec((1, 2 * HD_), lambda i: (0, 0)),
                  pl.BlockSpec((2 * HD_, HD_), lambda i: (0, 0)),
                  pl.BlockSpec((2 * HD_, HD_), lambda i: (0, 0))],
        out_specs=pl.BlockSpec((bn, D_), lambda i: (i, 0)),
        out_shape=jax.ShapeDtypeStruct((N_, D_), F32),
    )(p0, p1, b1, wl, wr)


def _tc_combine2(q0, q1, b2):
    bn = 1000

    def body(q0_ref, q1_ref, b2_ref, h_ref, hp_ref):
        acc = q0_ref[...] + q1_ref[...]
        hv = acc[:, :HD_] / (acc[:, HD_:HD_ + 1] + 1e-16)
        hv = jnp.maximum(hv + b2_ref[...], 0.0)
        h_ref[...] = hv
        hp_ref[...] = jnp.concatenate([hv, jnp.zeros((bn, D_ - HD_), F32)],
                                      axis=1)

    return pl.pallas_call(
        body,
        grid=(N_ // bn,),
        in_specs=[pl.BlockSpec((bn, D_), lambda i: (i, 0)),
                  pl.BlockSpec((bn, D_), lambda i: (i, 0)),
                  pl.BlockSpec((1, HD_), lambda i: (0, 0))],
        out_specs=(pl.BlockSpec((bn, HD_), lambda i: (i, 0)),
                   pl.BlockSpec((bn, D_), lambda i: (i, 0))),
        out_shape=(jax.ShapeDtypeStruct((N_, HD_), F32),
                   jax.ShapeDtypeStruct((N_, D_), F32)),
    )(q0, q1, b2)


def _tc_edge_mlp(hs, hd, ea, w3a, w3b, w3c, b3, w4, b4):
    bp = 2048

    def body(hs_ref, hd_ref, ea_ref, w3a_ref, w3b_ref, w3c_ref, b3_ref,
             w4_ref, b4_ref, o_ref):
        hid = (jnp.dot(hs_ref[0], w3a_ref[...], preferred_element_type=F32)
               + jnp.dot(hd_ref[0], w3b_ref[...], preferred_element_type=F32)
               + ea_ref[:, 0:1] * w3c_ref[0:1, :]
               + ea_ref[:, 1:2] * w3c_ref[1:2, :]
               + b3_ref[...])
        hid = jnp.maximum(hid, 0.0)
        o_ref[...] = jnp.dot(hid, w4_ref[...],
                             preferred_element_type=F32) + b4_ref[...]

    return pl.pallas_call(
        body,
        grid=(PP // bp,),
        in_specs=[pl.BlockSpec((1, bp, D_), lambda i: (0, i, 0)),
                  pl.BlockSpec((1, bp, D_), lambda i: (1, i, 0)),
                  pl.BlockSpec((bp, 2), lambda i: (i, 0)),
                  pl.BlockSpec((D_, HD_), lambda i: (0, 0)),
                  pl.BlockSpec((D_, HD_), lambda i: (0, 0)),
                  pl.BlockSpec((2, HD_), lambda i: (0, 0)),
                  pl.BlockSpec((1, HD_), lambda i: (0, 0)),
                  pl.BlockSpec((HD_, 1), lambda i: (0, 0)),
                  pl.BlockSpec((1, 1), lambda i: (0, 0))],
        out_specs=pl.BlockSpec((bp, 1), lambda i: (i, 0)),
        out_shape=jax.ShapeDtypeStruct((PP, 1), F32),
    )(hs, hd, ea, w3a, w3b, w3c, b3, w4, b4)


@jax.jit
def kernel(x, edge_index, edge_pairs, edge_attr, Wl1, Wr1, att1, b1,
           Wl2, Wr2, att2, b2, W3, b3, W4, b4):
    ei = edge_index.astype(I32)
    loop_idx = jnp.arange(N_, dtype=I32)
    epad = jnp.zeros((EP - E1,), I32)
    srcp = jnp.concatenate([ei[0], loop_idx, epad])
    dstp = jnp.concatenate([ei[1], loop_idx, epad])
    zeros = jnp.zeros((ZR, D_), F32)

    istk = jnp.stack([srcp, dstp])
    padrows = jnp.zeros((NP - N_, D_), F32)

    xl, xr = _tc_mm2(x, Wl1, Wr1)
    t1 = jnp.stack([jnp.concatenate([xl, padrows]),
                    jnp.concatenate([xr, padrows])])
    ab1 = _sc_gather_stk(t1, istk, EP)
    m1 = _tc_score1(ab1, ab1, att1.reshape(1, 2 * HD_))
    parts1 = _sc_scatter_headsplit(m1, dstp, zeros)
    t2 = _tc_combine1(parts1[:N_], parts1[NP:NP + N_],
                      b1.reshape(1, 2 * HD_), Wl2, Wr2)

    t2p = jnp.concatenate([t2, padrows])
    ab2 = _sc_gather_stk(jnp.stack([t2p, t2p]), istk, EP)
    m2 = _tc_score2(ab2, ab2, att2.reshape(1, HD_))
    parts2 = _sc_scatter_half(m2, dstp, zeros)
    h, hp = _tc_combine2(parts2[:N_], parts2[NP:NP + N_], b2.reshape(1, HD_))

    ep = edge_pairs.astype(I32)
    ppad = jnp.zeros((PP - P_,), I32)
    spp = jnp.concatenate([ep[0], ppad])
    dpp = jnp.concatenate([ep[1], ppad])
    eap = jnp.concatenate([edge_attr, jnp.zeros((PP - P_, 2), F32)], axis=0)
    hpp = jnp.concatenate([hp, padrows])
    hsd = _sc_gather_stk(jnp.stack([hpp, hpp]), jnp.stack([spp, dpp]), PP)
    hs = hdg = hsd
    zw = jnp.zeros((HD_, HD_), F32)
    w3a = jnp.concatenate([W3[:HD_], zw], axis=0)
    w3b = jnp.concatenate([W3[HD_:2 * HD_], zw], axis=0)
    out = _tc_edge_mlp(hs, hdg, eap, w3a, w3b, W3[2 * HD_:],
                       b3.reshape(1, HD_), W4, b4.reshape(1, 1))
    return (out[:P_, 0], h)


# speedup vs baseline: 2.1466x; 1.0807x over previous
import functools

import jax
import jax.numpy as jnp
from jax import lax
from jax.experimental import pallas as pl
from jax.experimental.pallas import tpu as pltpu
from jax.experimental.pallas import tpu_sc as plsc

F32 = jnp.float32
I32 = jnp.int32

N_ = 10000
E_ = 320000
P_ = 200000
HD_ = 64
E1 = E_ + N_

NC, NS = 2, 16
NW = NC * NS
CHUNK = 128
D_ = 128

EC = 82
EW = CHUNK * EC
EP = NW * EW
EC2 = 2 * EC
EW2 = CHUNK * EC2

PC = 50
PW = CHUNK * PC
PP = NW * PW

NP = 10240
RT = NP // NS
ZR = 128


def _mesh():
    return plsc.VectorSubcoreMesh(core_axis_name="c", subcore_axis_name="s")


def _sc_gather_stk(tstk, istk, total):
    per_tile = total // NS
    n_chunks = per_tile // CHUNK

    @functools.partial(
        pl.kernel,
        out_type=jax.ShapeDtypeStruct((2, total, D_), F32),
        mesh=_mesh(),
        scratch_types=[
            pltpu.VMEM((2, CHUNK), I32),
            pltpu.VMEM((2, CHUNK, D_), F32),
            pltpu.VMEM_SHARED((NP, D_), F32),
        ] + [pltpu.SemaphoreType.DMA] * 6,
    )
    def k(t_hbm, i_hbm, out, i_v, r_v, tab, *sems):
        si, sg, st = sems[0:2], sems[2:4], sems[4:6]
        c = lax.axis_index("c")
        s = lax.axis_index("s")
        pltpu.sync_copy(t_hbm.at[c, pl.ds(s * RT, RT)],
                        tab.at[pl.ds(s * RT, RT)])
        plsc.subcore_barrier()

        base = s * per_tile
        for b in range(2):
            off = base + b * CHUNK
            pltpu.async_copy(i_hbm.at[c, pl.ds(off, CHUNK)], i_v.at[b], si[b])

        @pl.loop(0, n_chunks, step=2)
        def _(i0):
            for b in range(2):
                o = 1 - b
                off = base + (i0 + b) * CHUNK
                pltpu.make_async_copy(i_hbm.at[c, pl.ds(off, CHUNK)],
                                      i_v.at[b], si[b]).wait()

                def _free():
                    pltpu.make_async_copy(r_v.at[b],
                                          out.at[c, pl.ds(off, CHUNK)],
                                          st[b]).wait()
                pl.when(i0 > 0)(_free)

                pltpu.async_copy(tab.at[i_v.at[b]], r_v.at[b], sg[b])

                def _finish():
                    poff = off - CHUNK
                    pltpu.make_async_copy(tab.at[i_v.at[o]], r_v.at[o],
                                          sg[o]).wait()

                    def _prefetch():
                        noff = off + CHUNK
                        pltpu.async_copy(i_hbm.at[c, pl.ds(noff, CHUNK)],
                                         i_v.at[o], si[o])
                    if b == 0:
                        _prefetch()
                    else:
                        pl.when(i0 < n_chunks - 2)(_prefetch)
                    pltpu.async_copy(r_v.at[o], out.at[c, pl.ds(poff, CHUNK)],
                                     st[o])
                if b == 1:
                    _finish()
                else:
                    pl.when(i0 > 0)(_finish)

        loff = base + (n_chunks - 1) * CHUNK
        pltpu.make_async_copy(tab.at[i_v.at[1]], r_v.at[1], sg[1]).wait()
        pltpu.async_copy(r_v.at[1], out.at[c, pl.ds(loff, CHUNK)], st[1])
        for b in range(2):
            off = base + (n_chunks - 2 + b) * CHUNK
            pltpu.make_async_copy(r_v.at[b], out.at[c, pl.ds(off, CHUNK)],
                                  st[b]).wait()

    return k(tstk, istk)


def _scatter_body(m_slice_fn, d_hbm, out, m_v, d_v, acc, sems,
                  z_hbm, base, n_chunks, c, s):
    sdm, smm, ssc = sems[0:2], sems[2:4], sems[4:6]

    for j in range(RT // ZR):
        pltpu.sync_copy(z_hbm, acc.at[pl.ds(s * RT + j * ZR, ZR)])
    plsc.subcore_barrier()

    for b in range(2):
        off = base + b * CHUNK
        pltpu.async_copy(d_hbm.at[pl.ds(off, CHUNK)], d_v.at[b], sdm[b])
        pltpu.async_copy(m_slice_fn(off), m_v.at[b], smm[b])

    @pl.loop(0, n_chunks, step=2)
    def _(i0):
        for b in range(2):
            o = 1 - b
            off = base + (i0 + b) * CHUNK
            pltpu.make_async_copy(d_hbm.at[pl.ds(off, CHUNK)],
                                  d_v.at[b], sdm[b]).wait()
            pltpu.make_async_copy(m_slice_fn(off), m_v.at[b], smm[b]).wait()
            pltpu.async_copy(m_v.at[b], acc.at[d_v.at[b]], ssc[b], add=True)

            def _finish():
                pltpu.make_async_copy(m_v.at[o], acc.at[d_v.at[o]],
                                      ssc[o]).wait()
                def _prefetch():
                    noff = off + CHUNK
                    pltpu.async_copy(d_hbm.at[pl.ds(noff, CHUNK)],
                                     d_v.at[o], sdm[o])
                    pltpu.async_copy(m_slice_fn(noff), m_v.at[o], smm[o])
                if b == 0:
                    _prefetch()
                else:
                    pl.when(i0 < n_chunks - 2)(_prefetch)
            if b == 1:
                _finish()
            else:
                pl.when(i0 > 0)(_finish)

    pltpu.make_async_copy(m_v.at[1], acc.at[d_v.at[1]], ssc[1]).wait()
    plsc.subcore_barrier()
    pltpu.sync_copy(acc.at[pl.ds(s * RT, RT)],
                    out.at[pl.ds(c * NP + s * RT, RT)])


def _scatter_scratch():
    return [
        pltpu.VMEM((2, CHUNK, D_), F32),
        pltpu.VMEM((2, CHUNK), I32),
        pltpu.VMEM_SHARED((NP, D_), F32),
    ] + [pltpu.SemaphoreType.DMA] * 6


def _sc_scatter_headsplit(m2, dst, zeros):
    @functools.partial(
        pl.kernel,
        out_type=jax.ShapeDtypeStruct((2 * NP, D_), F32),
        mesh=_mesh(),
        scratch_types=_scatter_scratch(),
    )
    def k(m_hbm, d_hbm, z_hbm, out, m_v, d_v, acc, *sems):
        c = lax.axis_index("c")
        s = lax.axis_index("s")
        base = s * EW2
        _scatter_body(lambda off: m_hbm.at[c, pl.ds(off, CHUNK)],
                      d_hbm, out, m_v, d_v, acc, sems,
                      z_hbm, base, EC2, c, s)

    return k(m2, dst, zeros)


def _sc_scatter_half(m, dst, zeros):
    @functools.partial(
        pl.kernel,
        out_type=jax.ShapeDtypeStruct((2 * NP, D_), F32),
        mesh=_mesh(),
        scratch_types=_scatter_scratch(),
    )
    def k(m_hbm, d_hbm, z_hbm, out, m_v, d_v, acc, *sems):
        c = lax.axis_index("c")
        s = lax.axis_index("s")
        base = (s * NC + c) * EW
        _scatter_body(lambda off: m_hbm.at[pl.ds(off, CHUNK)],
                      d_hbm, out, m_v, d_v, acc, sems,
                      z_hbm, base, EC, c, s)

    return k(m, dst, zeros)


def _tc_mm2(x, wa, wb):
    n, din = x.shape
    dout = wa.shape[1]
    bn = 1000

    def body(x_ref, wa_ref, wb_ref, oa_ref, ob_ref):
        xv = x_ref[...]
        oa_ref[...] = jnp.dot(xv, wa_ref[...], preferred_element_type=F32)
        ob_ref[...] = jnp.dot(xv, wb_ref[...], preferred_element_type=F32)

    return pl.pallas_call(
        body,
        grid=(n // bn,),
        in_specs=[pl.BlockSpec((bn, din), lambda i: (i, 0)),
                  pl.BlockSpec((din, dout), lambda i: (0, 0)),
                  pl.BlockSpec((din, dout), lambda i: (0, 0))],
        out_specs=(pl.BlockSpec((bn, dout), lambda i: (i, 0)),
                   pl.BlockSpec((bn, dout), lambda i: (i, 0))),
        out_shape=(jax.ShapeDtypeStruct((n, dout), F32),
                   jax.ShapeDtypeStruct((n, dout), F32)),
    )(x, wa, wb)


def _tc_score1(a, b, att_flat):
    be = 4096

    def body(a_ref, b_ref, att_ref, m_ref):
        i = pl.program_id(0)
        av = a_ref[0]
        e = av + b_ref[0]
        e = jnp.maximum(e, 0.2 * e)
        ea = e * att_ref[...]
        s0 = jnp.sum(ea[:, :HD_], axis=1, keepdims=True)
        s1 = jnp.sum(ea[:, HD_:], axis=1, keepdims=True)
        rid = i * be + lax.broadcasted_iota(I32, (be, 1), 0)
        mask = (rid < E1).astype(F32)
        x0 = jnp.exp(s0) * mask
        x1 = jnp.exp(s1) * mask
        z = jnp.zeros((be, D_ - HD_ - 1), F32)
        m_ref[0] = jnp.concatenate([av[:, :HD_] * x0, x0, z], axis=1)
        m_ref[1] = jnp.concatenate([av[:, HD_:] * x1, x1, z], axis=1)

    return pl.pallas_call(
        body,
        grid=(EP // be,),
        in_specs=[pl.BlockSpec((1, be, D_), lambda i: (0, i, 0)),
                  pl.BlockSpec((1, be, D_), lambda i: (1, i, 0)),
                  pl.BlockSpec((1, 2 * HD_), lambda i: (0, 0))],
        out_specs=pl.BlockSpec((2, be, D_), lambda i: (0, i, 0)),
        out_shape=jax.ShapeDtypeStruct((2, EP, D_), F32),
    )(a, b, att_flat)


def _tc_score2(a, b, att_flat):
    be = 4096

    def body(a_ref, b_ref, att_ref, m_ref):
        i = pl.program_id(0)
        av = a_ref[0][:, :HD_]
        e = av + b_ref[0][:, HD_:]
        e = jnp.maximum(e, 0.2 * e)
        s0 = jnp.sum(e * att_ref[...], axis=1, keepdims=True)
        rid = i * be + lax.broadcasted_iota(I32, (be, 1), 0)
        mask = (rid < E1).astype(F32)
        x0 = jnp.exp(s0) * mask
        m_ref[...] = jnp.concatenate(
            [av * x0, x0, jnp.zeros((be, D_ - HD_ - 1), F32)], axis=1)

    return pl.pallas_call(
        body,
        grid=(EP // be,),
        in_specs=[pl.BlockSpec((1, be, D_), lambda i: (0, i, 0)),
                  pl.BlockSpec((1, be, D_), lambda i: (1, i, 0)),
                  pl.BlockSpec((1, HD_), lambda i: (0, 0))],
        out_specs=pl.BlockSpec((be, D_), lambda i: (i, 0)),
        out_shape=jax.ShapeDtypeStruct((EP, D_), F32),
    )(a, b, att_flat)


def _tc_combine1(p0, p1, b1, wl, wr):
    bn = 1000

    def body(p0_ref, p1_ref, b1_ref, wl_ref, wr_ref, t_ref):
        pa = p0_ref[...]
        pb = p1_ref[...]
        h0 = pa[:, :HD_] / (pa[:, HD_:HD_ + 1] + 1e-16)
        h1 = pb[:, :HD_] / (pb[:, HD_:HD_ + 1] + 1e-16)
        h = jnp.maximum(jnp.concatenate([h0, h1], axis=1) + b1_ref[...], 0.0)
        t_ref[:, :HD_] = jnp.dot(h, wl_ref[...], preferred_element_type=F32)
        t_ref[:, HD_:] = jnp.dot(h, wr_ref[...], preferred_element_type=F32)

    return pl.pallas_call(
        body,
        grid=(N_ // bn,),
        in_specs=[pl.BlockSpec((bn, D_), lambda i: (i, 0)),
                  pl.BlockSpec((bn, D_), lambda i: (i, 0)),
                  pl.BlockSpec((1, 2 * HD_), lambda i: (0, 0)),
                  pl.BlockSpec((2 * HD_, HD_), lambda i: (0, 0)),
                  pl.BlockSpec((2 * HD_, HD_), lambda i: (0, 0))],
        out_specs=pl.BlockSpec((bn, D_), lambda i: (i, 0)),
        out_shape=jax.ShapeDtypeStruct((N_, D_), F32),
    )(p0, p1, b1, wl, wr)


def _tc_combine2(q0, q1, b2):
    bn = 1000

    def body(q0_ref, q1_ref, b2_ref, h_ref, hp_ref):
        acc = q0_ref[...] + q1_ref[...]
        hv = acc[:, :HD_] / (acc[:, HD_:HD_ + 1] + 1e-16)
        hv = jnp.maximum(hv + b2_ref[...], 0.0)
        h_ref[...] = hv
        hp_ref[...] = jnp.concatenate([hv, jnp.zeros((bn, D_ - HD_), F32)],
                                      axis=1)

    return pl.pallas_call(
        body,
        grid=(N_ // bn,),
        in_specs=[pl.BlockSpec((bn, D_), lambda i: (i, 0)),
                  pl.BlockSpec((bn, D_), lambda i: (i, 0)),
                  pl.BlockSpec((1, HD_), lambda i: (0, 0))],
        out_specs=(pl.BlockSpec((bn, HD_), lambda i: (i, 0)),
                   pl.BlockSpec((bn, D_), lambda i: (i, 0))),
        out_shape=(jax.ShapeDtypeStruct((N_, HD_), F32),
                   jax.ShapeDtypeStruct((N_, D_), F32)),
    )(q0, q1, b2)


def _tc_edge_mlp(hs, hd, ea, w3a, w3b, w3c, b3, w4, b4):
    bp = 4096

    def body(hs_ref, hd_ref, ea_ref, w3a_ref, w3b_ref, w3c_ref, b3_ref,
             w4_ref, b4_ref, o_ref):
        hid = (jnp.dot(hs_ref[0], w3a_ref[...], preferred_element_type=F32)
               + jnp.dot(hd_ref[0], w3b_ref[...], preferred_element_type=F32)
               + ea_ref[:, 0:1] * w3c_ref[0:1, :]
               + ea_ref[:, 1:2] * w3c_ref[1:2, :]
               + b3_ref[...])
        hid = jnp.maximum(hid, 0.0)
        o_ref[...] = jnp.dot(hid, w4_ref[...],
                             preferred_element_type=F32) + b4_ref[...]

    return pl.pallas_call(
        body,
        grid=(PP // bp,),
        in_specs=[pl.BlockSpec((1, bp, D_), lambda i: (0, i, 0)),
                  pl.BlockSpec((1, bp, D_), lambda i: (1, i, 0)),
                  pl.BlockSpec((bp, 2), lambda i: (i, 0)),
                  pl.BlockSpec((D_, HD_), lambda i: (0, 0)),
                  pl.BlockSpec((D_, HD_), lambda i: (0, 0)),
                  pl.BlockSpec((2, HD_), lambda i: (0, 0)),
                  pl.BlockSpec((1, HD_), lambda i: (0, 0)),
                  pl.BlockSpec((HD_, 1), lambda i: (0, 0)),
                  pl.BlockSpec((1, 1), lambda i: (0, 0))],
        out_specs=pl.BlockSpec((bp, 1), lambda i: (i, 0)),
        out_shape=jax.ShapeDtypeStruct((PP, 1), F32),
    )(hs, hd, ea, w3a, w3b, w3c, b3, w4, b4)


@jax.jit
def kernel(x, edge_index, edge_pairs, edge_attr, Wl1, Wr1, att1, b1,
           Wl2, Wr2, att2, b2, W3, b3, W4, b4):
    ei = edge_index.astype(I32)
    loop_idx = jnp.arange(N_, dtype=I32)
    epad = jnp.zeros((EP - E1,), I32)
    srcp = jnp.concatenate([ei[0], loop_idx, epad])
    dstp = jnp.concatenate([ei[1], loop_idx, epad])
    zeros = jnp.zeros((ZR, D_), F32)

    istk = jnp.stack([srcp, dstp])
    padrows = jnp.zeros((NP - N_, D_), F32)

    xl, xr = _tc_mm2(x, Wl1, Wr1)
    t1 = jnp.stack([jnp.concatenate([xl, padrows]),
                    jnp.concatenate([xr, padrows])])
    ab1 = _sc_gather_stk(t1, istk, EP)
    m1 = _tc_score1(ab1, ab1, att1.reshape(1, 2 * HD_))
    parts1 = _sc_scatter_headsplit(m1, dstp, zeros)
    t2 = _tc_combine1(parts1[:N_], parts1[NP:NP + N_],
                      b1.reshape(1, 2 * HD_), Wl2, Wr2)

    t2p = jnp.concatenate([t2, padrows])
    ab2 = _sc_gather_stk(jnp.stack([t2p, t2p]), istk, EP)
    m2 = _tc_score2(ab2, ab2, att2.reshape(1, HD_))
    parts2 = _sc_scatter_half(m2, dstp, zeros)
    h, hp = _tc_combine2(parts2[:N_], parts2[NP:NP + N_], b2.reshape(1, HD_))

    ep = edge_pairs.astype(I32)
    ppad = jnp.zeros((PP - P_,), I32)
    spp = jnp.concatenate([ep[0], ppad])
    dpp = jnp.concatenate([ep[1], ppad])
    eap = jnp.concatenate([edge_attr, jnp.zeros((PP - P_, 2), F32)], axis=0)
    hpp = jnp.concatenate([hp, padrows])
    hsd = _sc_gather_stk(jnp.stack([hpp, hpp]), jnp.stack([spp, dpp]), PP)
    hs = hdg = hsd
    zw = jnp.zeros((HD_, HD_), F32)
    w3a = jnp.concatenate([W3[:HD_], zw], axis=0)
    w3b = jnp.concatenate([W3[HD_:2 * HD_], zw], axis=0)
    out = _tc_edge_mlp(hs, hdg, eap, w3a, w3b, W3[2 * HD_:],
                       b3.reshape(1, HD_), W4, b4.reshape(1, 1))
    return (out[:P_, 0], h)


# speedup vs baseline: 2.2029x; 1.0262x over previous
import functools

import jax
import jax.numpy as jnp
from jax import lax
from jax.experimental import pallas as pl
from jax.experimental.pallas import tpu as pltpu
from jax.experimental.pallas import tpu_sc as plsc

F32 = jnp.float32
I32 = jnp.int32

N_ = 10000
E_ = 320000
P_ = 200000
HD_ = 64
E1 = E_ + N_

NC, NS = 2, 16
NW = NC * NS
CHUNK = 128
D_ = 128

EC = 82
EW = CHUNK * EC
EP = NW * EW
EC2 = 2 * EC
EW2 = CHUNK * EC2

PC = 50
PW = CHUNK * PC
PP = NW * PW

NP = 10240
RT = NP // NS
ZR = 128


def _mesh():
    return plsc.VectorSubcoreMesh(core_axis_name="c", subcore_axis_name="s")


def _sc_gather_stk(tstk, istk, total):
    per_tile = total // NS
    n_chunks = per_tile // CHUNK

    @functools.partial(
        pl.kernel,
        out_type=jax.ShapeDtypeStruct((2, total, D_), F32),
        mesh=_mesh(),
        scratch_types=[
            pltpu.VMEM((2, CHUNK), I32),
            pltpu.VMEM((2, CHUNK, D_), F32),
            pltpu.VMEM_SHARED((NP, D_), F32),
        ] + [pltpu.SemaphoreType.DMA] * 6,
    )
    def k(t_hbm, i_hbm, out, i_v, r_v, tab, *sems):
        si, sg, st = sems[0:2], sems[2:4], sems[4:6]
        c = lax.axis_index("c")
        s = lax.axis_index("s")
        pltpu.sync_copy(t_hbm.at[c, pl.ds(s * RT, RT)],
                        tab.at[pl.ds(s * RT, RT)])
        plsc.subcore_barrier()

        base = s * per_tile
        for b in range(2):
            off = base + b * CHUNK
            pltpu.async_copy(i_hbm.at[c, pl.ds(off, CHUNK)], i_v.at[b], si[b])

        @pl.loop(0, n_chunks, step=2)
        def _(i0):
            for b in range(2):
                o = 1 - b
                off = base + (i0 + b) * CHUNK
                pltpu.make_async_copy(i_hbm.at[c, pl.ds(off, CHUNK)],
                                      i_v.at[b], si[b]).wait()

                def _free():
                    pltpu.make_async_copy(r_v.at[b],
                                          out.at[c, pl.ds(off, CHUNK)],
                                          st[b]).wait()
                pl.when(i0 > 0)(_free)

                pltpu.async_copy(tab.at[i_v.at[b]], r_v.at[b], sg[b])

                def _finish():
                    poff = off - CHUNK
                    pltpu.make_async_copy(tab.at[i_v.at[o]], r_v.at[o],
                                          sg[o]).wait()

                    def _prefetch():
                        noff = off + CHUNK
                        pltpu.async_copy(i_hbm.at[c, pl.ds(noff, CHUNK)],
                                         i_v.at[o], si[o])
                    if b == 0:
                        _prefetch()
                    else:
                        pl.when(i0 < n_chunks - 2)(_prefetch)
                    pltpu.async_copy(r_v.at[o], out.at[c, pl.ds(poff, CHUNK)],
                                     st[o])
                if b == 1:
                    _finish()
                else:
                    pl.when(i0 > 0)(_finish)

        loff = base + (n_chunks - 1) * CHUNK
        pltpu.make_async_copy(tab.at[i_v.at[1]], r_v.at[1], sg[1]).wait()
        pltpu.async_copy(r_v.at[1], out.at[c, pl.ds(loff, CHUNK)], st[1])
        for b in range(2):
            off = base + (n_chunks - 2 + b) * CHUNK
            pltpu.make_async_copy(r_v.at[b], out.at[c, pl.ds(off, CHUNK)],
                                  st[b]).wait()

    return k(tstk, istk)


def _scatter_body(m_slice_fn, d_hbm, out, m_v, d_v, acc, sems,
                  z_hbm, base, n_chunks, c, s):
    sdm, smm, ssc = sems[0:2], sems[2:4], sems[4:6]

    for j in range(RT // ZR):
        pltpu.sync_copy(z_hbm, acc.at[pl.ds(s * RT + j * ZR, ZR)])
    plsc.subcore_barrier()

    for b in range(2):
        off = base + b * CHUNK
        pltpu.async_copy(d_hbm.at[pl.ds(off, CHUNK)], d_v.at[b], sdm[b])
        pltpu.async_copy(m_slice_fn(off), m_v.at[b], smm[b])

    @pl.loop(0, n_chunks, step=2)
    def _(i0):
        for b in range(2):
            o = 1 - b
            off = base + (i0 + b) * CHUNK
            pltpu.make_async_copy(d_hbm.at[pl.ds(off, CHUNK)],
                                  d_v.at[b], sdm[b]).wait()
            pltpu.make_async_copy(m_slice_fn(off), m_v.at[b], smm[b]).wait()
            pltpu.async_copy(m_v.at[b], acc.at[d_v.at[b]], ssc[b], add=True)

            def _finish():
                pltpu.make_async_copy(m_v.at[o], acc.at[d_v.at[o]],
                                      ssc[o]).wait()
                def _prefetch():
                    noff = off + CHUNK
                    pltpu.async_copy(d_hbm.at[pl.ds(noff, CHUNK)],
                                     d_v.at[o], sdm[o])
                    pltpu.async_copy(m_slice_fn(noff), m_v.at[o], smm[o])
                if b == 0:
                    _prefetch()
                else:
                    pl.when(i0 < n_chunks - 2)(_prefetch)
            if b == 1:
                _finish()
            else:
                pl.when(i0 > 0)(_finish)

    pltpu.make_async_copy(m_v.at[1], acc.at[d_v.at[1]], ssc[1]).wait()
    plsc.subcore_barrier()
    pltpu.sync_copy(acc.at[pl.ds(s * RT, RT)],
                    out.at[pl.ds(c * NP + s * RT, RT)])


def _scatter_scratch():
    return [
        pltpu.VMEM((2, CHUNK, D_), F32),
        pltpu.VMEM((2, CHUNK), I32),
        pltpu.VMEM_SHARED((NP, D_), F32),
    ] + [pltpu.SemaphoreType.DMA] * 6


def _sc_scatter_headsplit(m2, dst, zeros):
    @functools.partial(
        pl.kernel,
        out_type=jax.ShapeDtypeStruct((2 * NP, D_), F32),
        mesh=_mesh(),
        scratch_types=_scatter_scratch(),
    )
    def k(m_hbm, d_hbm, z_hbm, out, m_v, d_v, acc, *sems):
        c = lax.axis_index("c")
        s = lax.axis_index("s")
        base = s * EW2
        _scatter_body(lambda off: m_hbm.at[c, pl.ds(off, CHUNK)],
                      d_hbm, out, m_v, d_v, acc, sems,
                      z_hbm, base, EC2, c, s)

    return k(m2, dst, zeros)


def _sc_scatter_half(m, dst, zeros):
    @functools.partial(
        pl.kernel,
        out_type=jax.ShapeDtypeStruct((2 * NP, D_), F32),
        mesh=_mesh(),
        scratch_types=_scatter_scratch(),
    )
    def k(m_hbm, d_hbm, z_hbm, out, m_v, d_v, acc, *sems):
        c = lax.axis_index("c")
        s = lax.axis_index("s")
        base = (s * NC + c) * EW
        _scatter_body(lambda off: m_hbm.at[pl.ds(off, CHUNK)],
                      d_hbm, out, m_v, d_v, acc, sems,
                      z_hbm, base, EC, c, s)

    return k(m, dst, zeros)


def _tc_mm2(x, wa, wb):
    n, din = x.shape
    dout = wa.shape[1]
    bn = 1000

    def body(x_ref, wa_ref, wb_ref, oa_ref, ob_ref):
        xv = x_ref[...]
        oa_ref[...] = jnp.dot(xv, wa_ref[...], preferred_element_type=F32)
        ob_ref[...] = jnp.dot(xv, wb_ref[...], preferred_element_type=F32)

    return pl.pallas_call(
        body,
        grid=(n // bn,),
        in_specs=[pl.BlockSpec((bn, din), lambda i: (i, 0)),
                  pl.BlockSpec((din, dout), lambda i: (0, 0)),
                  pl.BlockSpec((din, dout), lambda i: (0, 0))],
        out_specs=(pl.BlockSpec((bn, dout), lambda i: (i, 0)),
                   pl.BlockSpec((bn, dout), lambda i: (i, 0))),
        out_shape=(jax.ShapeDtypeStruct((n, dout), F32),
                   jax.ShapeDtypeStruct((n, dout), F32)),
    )(x, wa, wb)


def _tc_score1(a, b, att_flat):
    be = 8192

    def body(a_ref, b_ref, att_ref, m_ref):
        i = pl.program_id(0)
        av = a_ref[0]
        e = av + b_ref[0]
        e = jnp.maximum(e, 0.2 * e)
        ea = e * att_ref[...]
        s0 = jnp.sum(ea[:, :HD_], axis=1, keepdims=True)
        s1 = jnp.sum(ea[:, HD_:], axis=1, keepdims=True)
        rid = i * be + lax.broadcasted_iota(I32, (be, 1), 0)
        mask = (rid < E1).astype(F32)
        x0 = jnp.exp(s0) * mask
        x1 = jnp.exp(s1) * mask
        z = jnp.zeros((be, D_ - HD_ - 1), F32)
        m_ref[0] = jnp.concatenate([av[:, :HD_] * x0, x0, z], axis=1)
        m_ref[1] = jnp.concatenate([av[:, HD_:] * x1, x1, z], axis=1)

    return pl.pallas_call(
        body,
        grid=(EP // be,),
        in_specs=[pl.BlockSpec((1, be, D_), lambda i: (0, i, 0)),
                  pl.BlockSpec((1, be, D_), lambda i: (1, i, 0)),
                  pl.BlockSpec((1, 2 * HD_), lambda i: (0, 0))],
        out_specs=pl.BlockSpec((2, be, D_), lambda i: (0, i, 0)),
        out_shape=jax.ShapeDtypeStruct((2, EP, D_), F32),
    )(a, b, att_flat)


def _tc_score2(a, b, att_flat):
    be = 8192

    def body(a_ref, b_ref, att_ref, m_ref):
        i = pl.program_id(0)
        av = a_ref[0][:, :HD_]
        e = av + b_ref[0][:, HD_:]
        e = jnp.maximum(e, 0.2 * e)
        s0 = jnp.sum(e * att_ref[...], axis=1, keepdims=True)
        rid = i * be + lax.broadcasted_iota(I32, (be, 1), 0)
        mask = (rid < E1).astype(F32)
        x0 = jnp.exp(s0) * mask
        m_ref[...] = jnp.concatenate(
            [av * x0, x0, jnp.zeros((be, D_ - HD_ - 1), F32)], axis=1)

    return pl.pallas_call(
        body,
        grid=(EP // be,),
        in_specs=[pl.BlockSpec((1, be, D_), lambda i: (0, i, 0)),
                  pl.BlockSpec((1, be, D_), lambda i: (1, i, 0)),
                  pl.BlockSpec((1, HD_), lambda i: (0, 0))],
        out_specs=pl.BlockSpec((be, D_), lambda i: (i, 0)),
        out_shape=jax.ShapeDtypeStruct((EP, D_), F32),
    )(a, b, att_flat)


def _tc_combine1(p0, p1, b1, wl, wr):
    bn = 1000

    def body(p0_ref, p1_ref, b1_ref, wl_ref, wr_ref, t_ref):
        pa = p0_ref[...]
        pb = p1_ref[...]
        h0 = pa[:, :HD_] / (pa[:, HD_:HD_ + 1] + 1e-16)
        h1 = pb[:, :HD_] / (pb[:, HD_:HD_ + 1] + 1e-16)
        h = jnp.maximum(jnp.concatenate([h0, h1], axis=1) + b1_ref[...], 0.0)
        t_ref[:, :HD_] = jnp.dot(h, wl_ref[...], preferred_element_type=F32)
        t_ref[:, HD_:] = jnp.dot(h, wr_ref[...], preferred_element_type=F32)

    return pl.pallas_call(
        body,
        grid=(N_ // bn,),
        in_specs=[pl.BlockSpec((bn, D_), lambda i: (i, 0)),
                  pl.BlockSpec((bn, D_), lambda i: (i, 0)),
                  pl.BlockSpec((1, 2 * HD_), lambda i: (0, 0)),
                  pl.BlockSpec((2 * HD_, HD_), lambda i: (0, 0)),
                  pl.BlockSpec((2 * HD_, HD_), lambda i: (0, 0))],
        out_specs=pl.BlockSpec((bn, D_), lambda i: (i, 0)),
        out_shape=jax.ShapeDtypeStruct((N_, D_), F32),
    )(p0, p1, b1, wl, wr)


def _tc_combine2(q0, q1, b2):
    bn = 1000

    def body(q0_ref, q1_ref, b2_ref, h_ref, hp_ref):
        acc = q0_ref[...] + q1_ref[...]
        hv = acc[:, :HD_] / (acc[:, HD_:HD_ + 1] + 1e-16)
        hv = jnp.maximum(hv + b2_ref[...], 0.0)
        h_ref[...] = hv
        hp_ref[...] = jnp.concatenate([hv, jnp.zeros((bn, D_ - HD_), F32)],
                                      axis=1)

    return pl.pallas_call(
        body,
        grid=(N_ // bn,),
        in_specs=[pl.BlockSpec((bn, D_), lambda i: (i, 0)),
                  pl.BlockSpec((bn, D_), lambda i: (i, 0)),
                  pl.BlockSpec((1, HD_), lambda i: (0, 0))],
        out_specs=(pl.BlockSpec((bn, HD_), lambda i: (i, 0)),
                   pl.BlockSpec((bn, D_), lambda i: (i, 0))),
        out_shape=(jax.ShapeDtypeStruct((N_, HD_), F32),
                   jax.ShapeDtypeStruct((N_, D_), F32)),
    )(q0, q1, b2)


def _tc_edge_mlp(hs, hd, ea, w3a, w3b, w3c, b3, w4, b4):
    bp = 8192

    def body(hs_ref, hd_ref, ea_ref, w3a_ref, w3b_ref, w3c_ref, b3_ref,
             w4_ref, b4_ref, o_ref):
        hid = (jnp.dot(hs_ref[0], w3a_ref[...], preferred_element_type=F32)
               + jnp.dot(hd_ref[0], w3b_ref[...], preferred_element_type=F32)
               + ea_ref[:, 0:1] * w3c_ref[0:1, :]
               + ea_ref[:, 1:2] * w3c_ref[1:2, :]
               + b3_ref[...])
        hid = jnp.maximum(hid, 0.0)
        o_ref[...] = jnp.dot(hid, w4_ref[...],
                             preferred_element_type=F32) + b4_ref[...]

    return pl.pallas_call(
        body,
        grid=(PP // bp,),
        in_specs=[pl.BlockSpec((1, bp, D_), lambda i: (0, i, 0)),
                  pl.BlockSpec((1, bp, D_), lambda i: (1, i, 0)),
                  pl.BlockSpec((bp, 2), lambda i: (i, 0)),
                  pl.BlockSpec((D_, HD_), lambda i: (0, 0)),
                  pl.BlockSpec((D_, HD_), lambda i: (0, 0)),
                  pl.BlockSpec((2, HD_), lambda i: (0, 0)),
                  pl.BlockSpec((1, HD_), lambda i: (0, 0)),
                  pl.BlockSpec((HD_, 1), lambda i: (0, 0)),
                  pl.BlockSpec((1, 1), lambda i: (0, 0))],
        out_specs=pl.BlockSpec((bp, 1), lambda i: (i, 0)),
        out_shape=jax.ShapeDtypeStruct((PP, 1), F32),
    )(hs, hd, ea, w3a, w3b, w3c, b3, w4, b4)


@jax.jit
def kernel(x, edge_index, edge_pairs, edge_attr, Wl1, Wr1, att1, b1,
           Wl2, Wr2, att2, b2, W3, b3, W4, b4):
    ei = edge_index.astype(I32)
    loop_idx = jnp.arange(N_, dtype=I32)
    epad = jnp.zeros((EP - E1,), I32)
    srcp = jnp.concatenate([ei[0], loop_idx, epad])
    dstp = jnp.concatenate([ei[1], loop_idx, epad])
    zeros = jnp.zeros((ZR, D_), F32)

    istk = jnp.stack([srcp, dstp])
    padrows = jnp.zeros((NP - N_, D_), F32)

    xl, xr = _tc_mm2(x, Wl1, Wr1)
    t1 = jnp.stack([jnp.concatenate([xl, padrows]),
                    jnp.concatenate([xr, padrows])])
    ab1 = _sc_gather_stk(t1, istk, EP)
    m1 = _tc_score1(ab1, ab1, att1.reshape(1, 2 * HD_))
    parts1 = _sc_scatter_headsplit(m1, dstp, zeros)
    t2 = _tc_combine1(parts1[:N_], parts1[NP:NP + N_],
                      b1.reshape(1, 2 * HD_), Wl2, Wr2)

    t2p = jnp.concatenate([t2, padrows])
    ab2 = _sc_gather_stk(jnp.stack([t2p, t2p]), istk, EP)
    m2 = _tc_score2(ab2, ab2, att2.reshape(1, HD_))
    parts2 = _sc_scatter_half(m2, dstp, zeros)
    h, hp = _tc_combine2(parts2[:N_], parts2[NP:NP + N_], b2.reshape(1, HD_))

    ep = edge_pairs.astype(I32)
    ppad = jnp.zeros((PP - P_,), I32)
    spp = jnp.concatenate([ep[0], ppad])
    dpp = jnp.concatenate([ep[1], ppad])
    eap = jnp.concatenate([edge_attr, jnp.zeros((PP - P_, 2), F32)], axis=0)
    hpp = jnp.concatenate([hp, padrows])
    hsd = _sc_gather_stk(jnp.stack([hpp, hpp]), jnp.stack([spp, dpp]), PP)
    hs = hdg = hsd
    zw = jnp.zeros((HD_, HD_), F32)
    w3a = jnp.concatenate([W3[:HD_], zw], axis=0)
    w3b = jnp.concatenate([W3[HD_:2 * HD_], zw], axis=0)
    out = _tc_edge_mlp(hs, hdg, eap, w3a, w3b, W3[2 * HD_:],
                       b3.reshape(1, HD_), W4, b4.reshape(1, 1))
    return (out[:P_, 0], h)
